# Initial kernel scaffold; baseline (speedup 1.0000x reference)
import jax
import jax.numpy as jnp
from jax.experimental import pallas as pl


def kernel(nodes, feat_data, adj1, adj2, adj3, prior, W_mlp, b_mlp, alpha1, alpha2, W2, b2, W3, b3):
    raise NotImplementedError("write your pallas kernel here")


# speedup vs baseline: 2.9346x; 2.9346x over previous
import functools

import jax
import jax.numpy as jnp
from jax import lax
from jax.experimental import pallas as pl
from jax.experimental.pallas import tpu as pltpu
from jax.experimental.pallas import tpu_sc as plsc

N = 10000
DEG = 16
DFEAT = 128
B = 1024
E1 = 64
E2 = 128
R3 = 3 * DEG

NC = 2
NS = 16
NW = NC * NS
NPAD = 10240

L1_PER_W = NPAD // NW
L1_CH = 10
L1_NCH = L1_PER_W // L1_CH
L2_PER_W = B // NW
L2_CH = 8
L2_NCH = L2_PER_W // L2_CH


def _mesh():
    return plsc.VectorSubcoreMesh(core_axis_name="c", subcore_axis_name="s",
                                  num_cores=NC, num_subcores=NS)


_SC_PARAMS = pltpu.CompilerParams(use_tc_tiling_on_sc=False)


def _wid():
    return lax.axis_index("s") * NC + lax.axis_index("c")


def _emb_body(f_ref, w_ref, b_ref, o_ref):
    x = jnp.dot(f_ref[...], w_ref[...], preferred_element_type=jnp.float32)
    o_ref[...] = jnp.maximum(x + b_ref[...], 0.0)


def _emb(feat, W, b):
    blk = 1024
    return pl.pallas_call(
        _emb_body,
        grid=(NPAD // blk,),
        in_specs=[pl.BlockSpec((blk, DFEAT), lambda i: (i, 0)),
                  pl.BlockSpec((DFEAT, E1), lambda i: (0, 0)),
                  pl.BlockSpec((1, E1), lambda i: (0, 0))],
        out_specs=pl.BlockSpec((blk, E1), lambda i: (i, 0)),
        out_shape=jax.ShapeDtypeStruct((NPAD, E1), jnp.float32),
    )(feat, W, b)


def _agg_body(tbl_hbm, idx_hbm, self_hbm, w_hbm, out_hbm,
              idxall, selfall, rA, rB, outA, outB, w_v,
              semA, semB, semOA, semOB,
              *, E, per_w, ch, nch, self_is_slice):
    wid = _wid()
    nq = E // 16
    wbase = wid * per_w

    pltpu.sync_copy(w_hbm, w_v)
    pltpu.sync_copy(idx_hbm.at[pl.ds(wbase * R3, per_w * R3)], idxall)
    pltpu.sync_copy(self_hbm.at[pl.ds(wbase, per_w)], selfall)

    wa = [[w_v[pl.ds(r * E + q * 16, 16)] for q in range(nq)]
          for r in range(3)]
    wb = [[w_v[pl.ds(3 * E + r * E + q * 16, 16)] for q in range(nq)]
          for r in range(3)]

    def fire(c, rows, sem):
        return pltpu.async_copy(
            tbl_hbm.at[idxall.at[pl.ds(c * (ch * R3), ch * R3)]], rows, sem)

    def wait_g(rows, sem):
        pltpu.make_async_copy(tbl_hbm.at[pl.ds(0, ch * R3)], rows, sem).wait()

    def wait_o(outb, sem):
        pltpu.make_async_copy(outb, out_hbm.at[pl.ds(0, ch)], sem).wait()

    fire(0, rA, semA)
    fire(1, rB, semB)

    def compute(c, rows, outb):
        def node(i, carry):
            rb = i * R3
            accA = [None] * nq
            accB = [None] * nq
            for r in range(3):
                for q in range(nq):
                    sl = pl.ds(q * 16, 16)
                    s = rows[rb + r * DEG, sl]
                    for j in range(1, DEG):
                        s = s + rows[rb + r * DEG + j, sl]
                    m = s * (1.0 / DEG)
                    if r == 0:
                        accA[q] = wa[r][q] * m
                        accB[q] = wb[r][q] * m
                    else:
                        accA[q] = accA[q] + wa[r][q] * m
                        accB[q] = accB[q] + wb[r][q] * m
            for q in range(nq):
                sl = pl.ds(q * 16, 16)
                outb[i, sl] = accA[q]
                outb[i, pl.ds(E + q * 16, 16)] = selfall[c * ch + i, sl] - accB[q]
            return carry

        lax.fori_loop(0, ch, node, 0)

    ni = nch // 2

    def iteration(i, carry):
        c0 = 2 * i
        c1 = 2 * i + 1
        wait_g(rA, semA)

        @pl.when(i > 0)
        def _():
            wait_o(outA, semOA)

        compute(c0, rA, outA)
        pltpu.async_copy(outA, out_hbm.at[pl.ds(wbase + c0 * ch, ch)], semOA)

        @pl.when(i < ni - 1)
        def _():
            fire(c0 + 2, rA, semA)

        wait_g(rB, semB)

        @pl.when(i > 0)
        def _():
            wait_o(outB, semOB)

        compute(c1, rB, outB)
        pltpu.async_copy(outB, out_hbm.at[pl.ds(wbase + c1 * ch, ch)], semOB)

        @pl.when(i < ni - 1)
        def _():
            fire(c1 + 2, rB, semB)

        return carry

    lax.fori_loop(0, ni, iteration, 0)
    wait_o(outA, semOA)
    wait_o(outB, semOB)


def _agg_kernel(tbl, idxf, selfv, wv, *, E, total, per_w, ch, nch):
    body = functools.partial(_agg_body, E=E, per_w=per_w, ch=ch, nch=nch,
                             self_is_slice=True)
    f = functools.partial(
        pl.kernel,
        out_type=jax.ShapeDtypeStruct((total, 2 * E), jnp.float32),
        mesh=_mesh(),
        compiler_params=_SC_PARAMS,
        scratch_types=[
            pltpu.VMEM((per_w * R3,), jnp.int32),
            pltpu.VMEM((per_w, E), jnp.float32),
            pltpu.VMEM((ch * R3, E), jnp.float32),
            pltpu.VMEM((ch * R3, E), jnp.float32),
            pltpu.VMEM((ch, 2 * E), jnp.float32),
            pltpu.VMEM((ch, 2 * E), jnp.float32),
            pltpu.VMEM((6 * E,), jnp.float32),
            pltpu.SemaphoreType.DMA,
            pltpu.SemaphoreType.DMA,
            pltpu.SemaphoreType.DMA,
            pltpu.SemaphoreType.DMA,
        ],
    )(body)
    return f(tbl, idxf, selfv, wv)


def _bg_body(nodes_hbm, aall_hbm, emb_hbm, int1_hbm,
             ab_hbm, e0b_hbm, i1b_hbm,
             nd_v, ab_v, e0b_v, i1b_v,
             s1, s2, s3):
    wid = _wid()
    base = wid * L2_PER_W
    pltpu.sync_copy(nodes_hbm.at[pl.ds(base, L2_PER_W)], nd_v)
    c1 = pltpu.async_copy(aall_hbm.at[nd_v], ab_v, s1)
    c2 = pltpu.async_copy(emb_hbm.at[nd_v], e0b_v, s2)
    c3 = pltpu.async_copy(int1_hbm.at[nd_v], i1b_v, s3)
    c1.wait(); c2.wait(); c3.wait()
    pltpu.sync_copy(ab_v, ab_hbm.at[pl.ds(base, L2_PER_W)])
    pltpu.sync_copy(e0b_v, e0b_hbm.at[pl.ds(base, L2_PER_W)])
    pltpu.sync_copy(i1b_v, i1b_hbm.at[pl.ds(base, L2_PER_W)])


def _bgather(nodes, aall2d, emb0, inter1):
    f = functools.partial(
        pl.kernel,
        out_type=[jax.ShapeDtypeStruct((B, R3), jnp.int32),
                  jax.ShapeDtypeStruct((B, E1), jnp.float32),
                  jax.ShapeDtypeStruct((B, E2), jnp.float32)],
        mesh=_mesh(),
        compiler_params=_SC_PARAMS,
        scratch_types=[
            pltpu.VMEM((L2_PER_W,), jnp.int32),
            pltpu.VMEM((L2_PER_W, R3), jnp.int32),
            pltpu.VMEM((L2_PER_W, E1), jnp.float32),
            pltpu.VMEM((L2_PER_W, E2), jnp.float32),
            pltpu.SemaphoreType.DMA,
            pltpu.SemaphoreType.DMA,
            pltpu.SemaphoreType.DMA,
        ],
    )(_bg_body)
    return f(nodes, aall2d, emb0, inter1)


def _head_body(e_ref, i1_ref, i2_ref, w2a, w2b, w2c, b2_ref, lp_ref, w3_ref,
               b3_ref, o_ref):
    x = (jnp.dot(e_ref[...], w2a[...], preferred_element_type=jnp.float32)
         + jnp.dot(i1_ref[...], w2b[...], preferred_element_type=jnp.float32)
         + jnp.dot(i2_ref[...], w2c[...], preferred_element_type=jnp.float32)
         + b2_ref[...])
    x = jnp.where(x >= 0.0, x, 0.3 * x)
    x = x + lp_ref[...]
    y = jnp.dot(x, w3_ref[...], preferred_element_type=jnp.float32) + b3_ref[...]
    o_ref[...] = jax.nn.sigmoid(y)


def _head(e0b, i1b, inter2, W2, b2, prior, W3, b3):
    return pl.pallas_call(
        _head_body,
        out_shape=jax.ShapeDtypeStruct((B, 1), jnp.float32),
    )(e0b, i1b, inter2, W2[:E1], W2[E1:E1 + E2], W2[E1 + E2:],
      b2.reshape(1, 2), jnp.log(prior).reshape(1, 2), W3, b3.reshape(1, 1))


def kernel(nodes, feat_data, adj1, adj2, adj3, prior, W_mlp, b_mlp,
           alpha1, alpha2, W2, b2, W3, b3):
    emb0 = _emb(feat_data, W_mlp, b_mlp.reshape(1, E1))

    Wm1 = jax.nn.softmax(alpha1, axis=1)
    w1 = jnp.concatenate([Wm1[:E1].T.reshape(-1), Wm1[E1:].T.reshape(-1)])
    Wm2 = jax.nn.softmax(alpha2, axis=1)
    w2v = jnp.concatenate([Wm2[:E2].T.reshape(-1), Wm2[E2:].T.reshape(-1)])

    pad = ((0, NPAD - N), (0, 0))
    aall2d = jnp.concatenate(
        [jnp.pad(adj1, pad)[:, None, :], jnp.pad(adj2, pad)[:, None, :],
         jnp.pad(adj3, pad)[:, None, :]], axis=1).reshape(NPAD, R3)
    aallf = aall2d.reshape(-1)

    inter1 = _agg_kernel(emb0, aallf, emb0, w1, E=E1, total=NPAD,
                         per_w=L1_PER_W, ch=L1_CH, nch=L1_NCH)

    ab, e0b, i1b = _bgather(nodes, aall2d, emb0, inter1)

    inter2 = _agg_kernel(inter1, ab.reshape(-1), i1b, w2v, E=E2, total=B,
                         per_w=L2_PER_W, ch=L2_CH, nch=L2_NCH)

    return _head(e0b, i1b, inter2, W2, b2, prior, W3, b3)


# speedup vs baseline: 4.2471x; 1.4472x over previous
import functools

import jax
import jax.numpy as jnp
from jax import lax
from jax.experimental import pallas as pl
from jax.experimental.pallas import tpu as pltpu
from jax.experimental.pallas import tpu_sc as plsc

N = 10000
DEG = 16
DFEAT = 128
B = 1024
E1 = 64
E2 = 128
R4 = 3 * DEG + 1

NC = 2
NS = 16
NW = NC * NS
NPAD = 10240

L1_PER_W = NPAD // NW
L1_CH = 8
L1_NCH = L1_PER_W // L1_CH
L2_PER_W = B // NW
L2_CH = 8
L2_NCH = L2_PER_W // L2_CH


def _mesh():
    return plsc.VectorSubcoreMesh(core_axis_name="c", subcore_axis_name="s",
                                  num_cores=NC, num_subcores=NS)


_SC_PARAMS = pltpu.CompilerParams(use_tc_tiling_on_sc=False)


def _wid():
    return lax.axis_index("s") * NC + lax.axis_index("c")


def _emb_body(f_ref, w_ref, b_ref, o_ref):
    x = jnp.dot(f_ref[...], w_ref[...], preferred_element_type=jnp.float32)
    o_ref[...] = jnp.maximum(x + b_ref[...], 0.0)


def _emb(feat, W, b):
    blk = 1024
    return pl.pallas_call(
        _emb_body,
        grid=(NPAD // blk,),
        in_specs=[pl.BlockSpec((blk, DFEAT), lambda i: (i, 0)),
                  pl.BlockSpec((DFEAT, E1), lambda i: (0, 0)),
                  pl.BlockSpec((1, E1), lambda i: (0, 0))],
        out_specs=pl.BlockSpec((blk, E1), lambda i: (i, 0)),
        out_shape=jax.ShapeDtypeStruct((NPAD, E1), jnp.float32),
    )(feat, W, b)


def _agg_body(tbl_hbm, idx_hbm, w_hbm, out_hbm,
              idxall, spm, rA, rB, outA, outB, w_v,
              semA, semB, semOA, semOB,
              *, E, per_w, ch, nch, tbl_rows):
    wid = _wid()
    nq = E // 16
    wbase = wid * per_w

    if spm is not None:
        sid = lax.axis_index("s")
        trs = tbl_rows // NS
        pltpu.sync_copy(tbl_hbm.at[pl.ds(sid * trs, trs)],
                        spm.at[pl.ds(sid * trs, trs)])

    pltpu.sync_copy(w_hbm, w_v)
    pltpu.sync_copy(idx_hbm.at[pl.ds(wbase * R4, per_w * R4)], idxall)
    if spm is not None:
        plsc.subcore_barrier()
    tbl = tbl_hbm if spm is None else spm

    wa = [[w_v[pl.ds(r * E + q * 16, 16)] for q in range(nq)]
          for r in range(3)]
    wb = [[w_v[pl.ds(3 * E + r * E + q * 16, 16)] for q in range(nq)]
          for r in range(3)]

    def fire(c, rows, sem):
        return pltpu.async_copy(
            tbl.at[idxall.at[pl.ds(c * (ch * R4), ch * R4)]], rows, sem)

    def wait_g(rows, sem):
        pltpu.make_async_copy(tbl_hbm.at[pl.ds(0, ch * R4)], rows, sem).wait()

    def wait_o(outb, sem):
        pltpu.make_async_copy(outb, out_hbm.at[pl.ds(0, ch)], sem).wait()

    fire(0, rA, semA)
    fire(1, rB, semB)

    def compute(c, rows, outb):
        def node(i, carry):
            rb = i * R4
            accA = [None] * nq
            accB = [None] * nq
            for r in range(3):
                for q in range(nq):
                    sl = pl.ds(q * 16, 16)
                    s = rows[rb + r * DEG, sl]
                    for j in range(1, DEG):
                        s = s + rows[rb + r * DEG + j, sl]
                    m = s * (1.0 / DEG)
                    if r == 0:
                        accA[q] = wa[r][q] * m
                        accB[q] = wb[r][q] * m
                    else:
                        accA[q] = accA[q] + wa[r][q] * m
                        accB[q] = accB[q] + wb[r][q] * m
            for q in range(nq):
                sl = pl.ds(q * 16, 16)
                outb[i, sl] = accA[q]
                outb[i, pl.ds(E + q * 16, 16)] = rows[rb + 48, sl] - accB[q]
            return carry

        lax.fori_loop(0, ch, node, 0)

    ni = nch // 2

    def iteration(i, carry):
        c0 = 2 * i
        c1 = 2 * i + 1
        wait_g(rA, semA)

        @pl.when(i > 0)
        def _():
            wait_o(outA, semOA)

        compute(c0, rA, outA)
        pltpu.async_copy(outA, out_hbm.at[pl.ds(wbase + c0 * ch, ch)], semOA)

        @pl.when(i < ni - 1)
        def _():
            fire(c0 + 2, rA, semA)

        wait_g(rB, semB)

        @pl.when(i > 0)
        def _():
            wait_o(outB, semOB)

        compute(c1, rB, outB)
        pltpu.async_copy(outB, out_hbm.at[pl.ds(wbase + c1 * ch, ch)], semOB)

        @pl.when(i < ni - 1)
        def _():
            fire(c1 + 2, rB, semB)

        return carry

    lax.fori_loop(0, ni, iteration, 0)
    wait_o(outA, semOA)
    wait_o(outB, semOB)


def _agg_kernel(tbl, idxf, wv, *, E, total, per_w, ch, nch, stage):
    tbl_rows = tbl.shape[0]
    body = functools.partial(_agg_body, E=E, per_w=per_w, ch=ch, nch=nch,
                             tbl_rows=tbl_rows)
    f = functools.partial(
        pl.kernel,
        out_type=jax.ShapeDtypeStruct((total, 2 * E), jnp.float32),
        mesh=_mesh(),
        compiler_params=_SC_PARAMS,
        scratch_types=[
            pltpu.VMEM((per_w * R4,), jnp.int32),
            pltpu.VMEM_SHARED((tbl_rows, E), jnp.float32) if stage else None,
            pltpu.VMEM((ch * R4, E), jnp.float32),
            pltpu.VMEM((ch * R4, E), jnp.float32),
            pltpu.VMEM((ch, 2 * E), jnp.float32),
            pltpu.VMEM((ch, 2 * E), jnp.float32),
            pltpu.VMEM((6 * E,), jnp.float32),
            pltpu.SemaphoreType.DMA,
            pltpu.SemaphoreType.DMA,
            pltpu.SemaphoreType.DMA,
            pltpu.SemaphoreType.DMA,
        ],
    )(body)
    return f(tbl, idxf, wv)


def _bg_body(nodes_hbm, aall_hbm, emb_hbm, int1_hbm,
             ab_hbm, e0b_hbm, i1b_hbm,
             nd_v, ab_v, e0b_v, i1b_v,
             s1, s2, s3):
    wid = _wid()
    base = wid * L2_PER_W
    pltpu.sync_copy(nodes_hbm.at[pl.ds(base, L2_PER_W)], nd_v)
    c1 = pltpu.async_copy(aall_hbm.at[nd_v], ab_v, s1)
    c2 = pltpu.async_copy(emb_hbm.at[nd_v], e0b_v, s2)
    c3 = pltpu.async_copy(int1_hbm.at[nd_v], i1b_v, s3)
    c1.wait(); c2.wait(); c3.wait()
    pltpu.sync_copy(ab_v, ab_hbm.at[pl.ds(base, L2_PER_W)])
    pltpu.sync_copy(e0b_v, e0b_hbm.at[pl.ds(base, L2_PER_W)])
    pltpu.sync_copy(i1b_v, i1b_hbm.at[pl.ds(base, L2_PER_W)])


def _bgather(nodes, aall2d, emb0, inter1):
    f = functools.partial(
        pl.kernel,
        out_type=[jax.ShapeDtypeStruct((B, R4), jnp.int32),
                  jax.ShapeDtypeStruct((B, E1), jnp.float32),
                  jax.ShapeDtypeStruct((B, E2), jnp.float32)],
        mesh=_mesh(),
        compiler_params=_SC_PARAMS,
        scratch_types=[
            pltpu.VMEM((L2_PER_W,), jnp.int32),
            pltpu.VMEM((L2_PER_W, R4), jnp.int32),
            pltpu.VMEM((L2_PER_W, E1), jnp.float32),
            pltpu.VMEM((L2_PER_W, E2), jnp.float32),
            pltpu.SemaphoreType.DMA,
            pltpu.SemaphoreType.DMA,
            pltpu.SemaphoreType.DMA,
        ],
    )(_bg_body)
    return f(nodes, aall2d, emb0, inter1)


def _head_body(e_ref, i1_ref, i2_ref, w2a, w2b, w2c, b2_ref, lp_ref, w3_ref,
               b3_ref, o_ref):
    x = (jnp.dot(e_ref[...], w2a[...], preferred_element_type=jnp.float32)
         + jnp.dot(i1_ref[...], w2b[...], preferred_element_type=jnp.float32)
         + jnp.dot(i2_ref[...], w2c[...], preferred_element_type=jnp.float32)
         + b2_ref[...])
    x = jnp.where(x >= 0.0, x, 0.3 * x)
    x = x + lp_ref[...]
    y = jnp.dot(x, w3_ref[...], preferred_element_type=jnp.float32) + b3_ref[...]
    o_ref[...] = jax.nn.sigmoid(y)


def _head(e0b, i1b, inter2, W2, b2, prior, W3, b3):
    return pl.pallas_call(
        _head_body,
        out_shape=jax.ShapeDtypeStruct((B, 1), jnp.float32),
    )(e0b, i1b, inter2, W2[:E1], W2[E1:E1 + E2], W2[E1 + E2:],
      b2.reshape(1, 2), jnp.log(prior).reshape(1, 2), W3, b3.reshape(1, 1))


def kernel(nodes, feat_data, adj1, adj2, adj3, prior, W_mlp, b_mlp,
           alpha1, alpha2, W2, b2, W3, b3):
    emb0 = _emb(feat_data, W_mlp, b_mlp.reshape(1, E1))

    Wm1 = jax.nn.softmax(alpha1, axis=1)
    w1 = jnp.concatenate([Wm1[:E1].T.reshape(-1), Wm1[E1:].T.reshape(-1)])
    Wm2 = jax.nn.softmax(alpha2, axis=1)
    w2v = jnp.concatenate([Wm2[:E2].T.reshape(-1), Wm2[E2:].T.reshape(-1)])

    pad = ((0, NPAD - N), (0, 0))
    aall2d = jnp.concatenate(
        [jnp.pad(adj1, pad)[:, None, :], jnp.pad(adj2, pad)[:, None, :],
         jnp.pad(adj3, pad)[:, None, :]], axis=1).reshape(NPAD, 3 * DEG)
    aall2d = jnp.concatenate(
        [aall2d, jnp.arange(NPAD, dtype=jnp.int32)[:, None]], axis=1)
    aallf = aall2d.reshape(-1)

    inter1 = _agg_kernel(emb0, aallf, w1, E=E1, total=NPAD,
                         per_w=L1_PER_W, ch=L1_CH, nch=L1_NCH,
                         stage=True)

    ab, e0b, i1b = _bgather(nodes, aall2d, emb0, inter1)

    inter2 = _agg_kernel(inter1, ab.reshape(-1), w2v, E=E2, total=B,
                         per_w=L2_PER_W, ch=L2_CH, nch=L2_NCH,
                         stage=False)

    return _head(e0b, i1b, inter2, W2, b2, prior, W3, b3)


# speedup vs baseline: 9.0734x; 2.1364x over previous
import functools

import jax
import jax.numpy as jnp
from jax import lax
from jax.experimental import pallas as pl
from jax.experimental.pallas import tpu as pltpu
from jax.experimental.pallas import tpu_sc as plsc

N = 10000
DEG = 16
DFEAT = 128
B = 1024
E1 = 64
E2 = 128
R4 = 3 * DEG + 1

NC = 2
NS = 16
NW = NC * NS
NPAD = 10240

L1_PER_W = NPAD // NW
L1_CH = 8
L1_NCH = L1_PER_W // L1_CH
L2_PER_W = B // NW
L2_CH = 8
L2_NCH = L2_PER_W // L2_CH


def _mesh():
    return plsc.VectorSubcoreMesh(core_axis_name="c", subcore_axis_name="s",
                                  num_cores=NC, num_subcores=NS)


_SC_PARAMS = pltpu.CompilerParams(use_tc_tiling_on_sc=False)


def _wid():
    return lax.axis_index("s") * NC + lax.axis_index("c")


def _emb_body(f_ref, w_ref, b_ref, o_ref):
    x = jnp.dot(f_ref[...], w_ref[...], preferred_element_type=jnp.float32)
    o_ref[...] = jnp.maximum(x + b_ref[...], 0.0)


def _emb(feat, W, b):
    blk = 1024
    return pl.pallas_call(
        _emb_body,
        grid=(NPAD // blk,),
        in_specs=[pl.BlockSpec((blk, DFEAT), lambda i: (i, 0)),
                  pl.BlockSpec((DFEAT, E1), lambda i: (0, 0)),
                  pl.BlockSpec((1, E1), lambda i: (0, 0))],
        out_specs=pl.BlockSpec((blk, E1), lambda i: (i, 0)),
        out_shape=jax.ShapeDtypeStruct((NPAD, E1), jnp.float32),
    )(feat, W, b)


def _agg_body(tbl_hbm, idx_hbm, w_hbm, out_hbm,
              idxall, spm, rA, rB, outA, outB, w_v,
              semA, semB, semOA, semOB,
              *, E, per_w, ch, nch, tbl_rows):
    wid = _wid()
    nq = E // 16
    wbase = wid * per_w

    if spm is not None:
        sid = lax.axis_index("s")
        trs = tbl_rows // NS
        pltpu.sync_copy(tbl_hbm.at[pl.ds(sid * trs, trs)],
                        spm.at[pl.ds(sid * trs, trs)])

    pltpu.sync_copy(w_hbm, w_v)
    pltpu.sync_copy(idx_hbm.at[pl.ds(wbase * R4, per_w * R4)], idxall)
    if spm is not None:
        plsc.subcore_barrier()
    tbl = tbl_hbm if spm is None else spm

    wa = [[w_v[pl.ds(r * E + q * 16, 16)] for q in range(nq)]
          for r in range(3)]
    wb = [[w_v[pl.ds(3 * E + r * E + q * 16, 16)] for q in range(nq)]
          for r in range(3)]

    def fire(c, rows, sem):
        return pltpu.async_copy(
            tbl.at[idxall.at[pl.ds(c * (ch * R4), ch * R4)]], rows, sem)

    def wait_g(rows, sem):
        pltpu.make_async_copy(tbl_hbm.at[pl.ds(0, ch * R4)], rows, sem).wait()

    def wait_o(outb, sem):
        pltpu.make_async_copy(outb, out_hbm.at[pl.ds(0, ch)], sem).wait()

    fire(0, rA, semA)
    fire(1, rB, semB)

    def compute(c, rows, outb):
        def node(i, carry):
            rb = i * R4
            accA = [None] * nq
            accB = [None] * nq
            for r in range(3):
                for q in range(nq):
                    sl = pl.ds(q * 16, 16)
                    s = rows[rb + r * DEG, sl]
                    for j in range(1, DEG):
                        s = s + rows[rb + r * DEG + j, sl]
                    m = s * (1.0 / DEG)
                    if r == 0:
                        accA[q] = wa[r][q] * m
                        accB[q] = wb[r][q] * m
                    else:
                        accA[q] = accA[q] + wa[r][q] * m
                        accB[q] = accB[q] + wb[r][q] * m
            for q in range(nq):
                sl = pl.ds(q * 16, 16)
                outb[i, sl] = accA[q]
                outb[i, pl.ds(E + q * 16, 16)] = rows[rb + 48, sl] - accB[q]
            return carry

        lax.fori_loop(0, ch, node, 0)

    ni = nch // 2

    def iteration(i, carry):
        c0 = 2 * i
        c1 = 2 * i + 1
        wait_g(rA, semA)

        @pl.when(i > 0)
        def _():
            wait_o(outA, semOA)

        compute(c0, rA, outA)
        pltpu.async_copy(outA, out_hbm.at[pl.ds(wbase + c0 * ch, ch)], semOA)

        @pl.when(i < ni - 1)
        def _():
            fire(c0 + 2, rA, semA)

        wait_g(rB, semB)

        @pl.when(i > 0)
        def _():
            wait_o(outB, semOB)

        compute(c1, rB, outB)
        pltpu.async_copy(outB, out_hbm.at[pl.ds(wbase + c1 * ch, ch)], semOB)

        @pl.when(i < ni - 1)
        def _():
            fire(c1 + 2, rB, semB)

        return carry

    lax.fori_loop(0, ni, iteration, 0)
    wait_o(outA, semOA)
    wait_o(outB, semOB)


def _agg_kernel(tbl, idxf, wv, *, E, total, per_w, ch, nch, stage):
    tbl_rows = tbl.shape[0]
    body = functools.partial(_agg_body, E=E, per_w=per_w, ch=ch, nch=nch,
                             tbl_rows=tbl_rows)
    f = functools.partial(
        pl.kernel,
        out_type=jax.ShapeDtypeStruct((total, 2 * E), jnp.float32),
        mesh=_mesh(),
        compiler_params=_SC_PARAMS,
        scratch_types=[
            pltpu.VMEM((per_w * R4,), jnp.int32),
            pltpu.VMEM_SHARED((tbl_rows, E), jnp.float32) if stage else None,
            pltpu.VMEM((ch * R4, E), jnp.float32),
            pltpu.VMEM((ch * R4, E), jnp.float32),
            pltpu.VMEM((ch, 2 * E), jnp.float32),
            pltpu.VMEM((ch, 2 * E), jnp.float32),
            pltpu.VMEM((6 * E,), jnp.float32),
            pltpu.SemaphoreType.DMA,
            pltpu.SemaphoreType.DMA,
            pltpu.SemaphoreType.DMA,
            pltpu.SemaphoreType.DMA,
        ],
    )(body)
    return f(tbl, idxf, wv)


def _bg_body(nodes_hbm, aall_hbm, emb_hbm, int1_hbm,
             ab_hbm, e0b_hbm, i1b_hbm,
             nd_v, ab_v, e0b_v, i1b_v,
             s1, s2, s3):
    wid = _wid()
    base = wid * L2_PER_W
    pltpu.sync_copy(nodes_hbm.at[pl.ds(base, L2_PER_W)], nd_v)
    c1 = pltpu.async_copy(aall_hbm.at[nd_v], ab_v, s1)
    c2 = pltpu.async_copy(emb_hbm.at[nd_v], e0b_v, s2)
    c3 = pltpu.async_copy(int1_hbm.at[nd_v], i1b_v, s3)
    c1.wait(); c2.wait(); c3.wait()
    pltpu.sync_copy(ab_v, ab_hbm.at[pl.ds(base, L2_PER_W)])
    pltpu.sync_copy(e0b_v, e0b_hbm.at[pl.ds(base, L2_PER_W)])
    pltpu.sync_copy(i1b_v, i1b_hbm.at[pl.ds(base, L2_PER_W)])


def _bgather(nodes, aall2d, emb0, inter1):
    f = functools.partial(
        pl.kernel,
        out_type=[jax.ShapeDtypeStruct((B, 64), jnp.int32),
                  jax.ShapeDtypeStruct((B, E1), jnp.float32),
                  jax.ShapeDtypeStruct((B, E2), jnp.float32)],
        mesh=_mesh(),
        compiler_params=_SC_PARAMS,
        scratch_types=[
            pltpu.VMEM((L2_PER_W,), jnp.int32),
            pltpu.VMEM((L2_PER_W, 64), jnp.int32),
            pltpu.VMEM((L2_PER_W, E1), jnp.float32),
            pltpu.VMEM((L2_PER_W, E2), jnp.float32),
            pltpu.SemaphoreType.DMA,
            pltpu.SemaphoreType.DMA,
            pltpu.SemaphoreType.DMA,
        ],
    )(_bg_body)
    return f(nodes, aall2d, emb0, inter1)


def _head_body(e_ref, i1_ref, i2_ref, w2a, w2b, w2c, b2_ref, lp_ref, w3_ref,
               b3_ref, o_ref):
    x = (jnp.dot(e_ref[...], w2a[...], preferred_element_type=jnp.float32)
         + jnp.dot(i1_ref[...], w2b[...], preferred_element_type=jnp.float32)
         + jnp.dot(i2_ref[...], w2c[...], preferred_element_type=jnp.float32)
         + b2_ref[...])
    x = jnp.where(x >= 0.0, x, 0.3 * x)
    x = x + lp_ref[...]
    y = jnp.dot(x, w3_ref[...], preferred_element_type=jnp.float32) + b3_ref[...]
    o_ref[...] = jax.nn.sigmoid(y)


def _head(e0b, i1b, inter2, W2, b2, prior, W3, b3):
    return pl.pallas_call(
        _head_body,
        out_shape=jax.ShapeDtypeStruct((B, 1), jnp.float32),
    )(e0b, i1b, inter2, W2[:E1], W2[E1:E1 + E2], W2[E1 + E2:],
      b2.reshape(1, 2), jnp.log(prior).reshape(1, 2), W3, b3.reshape(1, 1))


def kernel(nodes, feat_data, adj1, adj2, adj3, prior, W_mlp, b_mlp,
           alpha1, alpha2, W2, b2, W3, b3):
    emb0 = _emb(feat_data, W_mlp, b_mlp.reshape(1, E1))

    Wm1 = jax.nn.softmax(alpha1, axis=1)
    w1 = jnp.concatenate([Wm1[:E1].T.reshape(-1), Wm1[E1:].T.reshape(-1)])
    Wm2 = jax.nn.softmax(alpha2, axis=1)
    w2v = jnp.concatenate([Wm2[:E2].T.reshape(-1), Wm2[E2:].T.reshape(-1)])

    pad = ((0, NPAD - N), (0, 0))
    aall2d = jnp.concatenate(
        [jnp.pad(adj1, pad)[:, None, :], jnp.pad(adj2, pad)[:, None, :],
         jnp.pad(adj3, pad)[:, None, :]], axis=1).reshape(NPAD, 3 * DEG)
    iota = jnp.arange(NPAD, dtype=jnp.int32)[:, None]
    aall49 = jnp.concatenate([aall2d, iota], axis=1)
    aallf = aall49.reshape(-1)
    aall64 = jnp.concatenate(
        [aall49, jnp.broadcast_to(iota, (NPAD, 15))], axis=1)

    inter1 = _agg_kernel(emb0, aallf, w1, E=E1, total=NPAD,
                         per_w=L1_PER_W, ch=L1_CH, nch=L1_NCH,
                         stage=True)

    ab, e0b, i1b = _bgather(nodes, aall64, emb0, inter1)

    inter2 = _agg_kernel(inter1, ab[:, :R4].reshape(-1), w2v, E=E2, total=B,
                         per_w=L2_PER_W, ch=L2_CH, nch=L2_NCH,
                         stage=False)

    return _head(e0b, i1b, inter2, W2, b2, prior, W3, b3)


# speedup vs baseline: 12.1917x; 1.3437x over previous
import functools

import jax
import jax.numpy as jnp
from jax import lax
from jax.experimental import pallas as pl
from jax.experimental.pallas import tpu as pltpu
from jax.experimental.pallas import tpu_sc as plsc

N = 10000
DEG = 16
DFEAT = 128
B = 1024
E1 = 64
E2 = 128
R4 = 3 * DEG + 1

NC = 2
NS = 16
NW = NC * NS
NPAD = 10240

L1_PER_W = NPAD // NW
L1_CH = 16
L1_NCH = L1_PER_W // L1_CH
L2_PER_W = B // NW
L2_CH = 8
L2_NCH = L2_PER_W // L2_CH

_PK = plsc.PackFormat.INTERLEAVED


def _mesh():
    return plsc.VectorSubcoreMesh(core_axis_name="c", subcore_axis_name="s",
                                  num_cores=NC, num_subcores=NS)


_SC_PARAMS = pltpu.CompilerParams(use_tc_tiling_on_sc=False,
                                  needs_layout_passes=False)


def _wid():
    return lax.axis_index("s") * NC + lax.axis_index("c")


def _perm_w(wcol):
    E = wcol.shape[0]
    return wcol.reshape(E // 32, 16, 2).transpose(0, 2, 1).reshape(-1)


def _emb_body(f_ref, w_ref, b_ref, o_ref):
    x = jnp.dot(f_ref[...], w_ref[...], preferred_element_type=jnp.float32)
    o_ref[...] = jnp.maximum(x + b_ref[...], 0.0).astype(jnp.bfloat16)


def _emb(feat, W, b):
    blk = 1024
    return pl.pallas_call(
        _emb_body,
        grid=(NPAD // blk,),
        in_specs=[pl.BlockSpec((blk, DFEAT), lambda i: (i, 0)),
                  pl.BlockSpec((DFEAT, E1), lambda i: (0, 0)),
                  pl.BlockSpec((1, E1), lambda i: (0, 0))],
        out_specs=pl.BlockSpec((blk, E1), lambda i: (i, 0)),
        out_shape=jax.ShapeDtypeStruct((NPAD, E1), jnp.bfloat16),
    )(feat, W, b)


def _agg_body(tbl_hbm, idx_hbm, w_hbm, out_hbm,
              idxall, spm, rA, rB, outA, outB, w_v,
              semA, semB, semOA, semOB,
              *, E, per_w, ch, nch, tbl_rows):
    wid = _wid()
    ng = E // 32
    wbase = wid * per_w

    sid = lax.axis_index("s")
    trs = tbl_rows // NS
    pltpu.sync_copy(tbl_hbm.at[pl.ds(sid * trs, trs)],
                    spm.at[pl.ds(sid * trs, trs)])

    pltpu.sync_copy(w_hbm, w_v)
    pltpu.sync_copy(idx_hbm.at[pl.ds(wbase * R4, per_w * R4)], idxall)
    plsc.subcore_barrier()

    def wslice(kind, r, h, par):
        return w_v[pl.ds(kind * 3 * E + r * E + h * 32 + par * 16, 16)]

    wa = [[(wslice(0, r, h, 0), wslice(0, r, h, 1)) for h in range(ng)]
          for r in range(3)]
    wb = [[(wslice(1, r, h, 0), wslice(1, r, h, 1)) for h in range(ng)]
          for r in range(3)]

    def fire(c, rows, sem):
        return pltpu.async_copy(
            spm.at[idxall.at[pl.ds(c * (ch * R4), ch * R4)]], rows, sem)

    def wait_g(rows, sem):
        pltpu.make_async_copy(tbl_hbm.at[pl.ds(0, ch * R4)], rows, sem).wait()

    def wait_o(outb, sem):
        pltpu.make_async_copy(outb, out_hbm.at[pl.ds(0, ch)], sem).wait()

    fire(0, rA, semA)
    fire(1, rB, semB)

    def compute(c, rows, outb):
        def node(i, carry):
            rb = i * R4
            accA = [None] * ng
            accB = [None] * ng
            for r in range(3):
                for h in range(ng):
                    sl = pl.ds(h * 32, 32)
                    se, so = plsc.unpack(rows[rb + r * DEG, sl], format=_PK)
                    for j in range(1, DEG):
                        xe, xo = plsc.unpack(rows[rb + r * DEG + j, sl],
                                             format=_PK)
                        se = se + xe
                        so = so + xo
                    me = se * (1.0 / DEG)
                    mo = so * (1.0 / DEG)
                    if r == 0:
                        accA[h] = [wa[r][h][0] * me, wa[r][h][1] * mo]
                        accB[h] = [wb[r][h][0] * me, wb[r][h][1] * mo]
                    else:
                        accA[h][0] = accA[h][0] + wa[r][h][0] * me
                        accA[h][1] = accA[h][1] + wa[r][h][1] * mo
                        accB[h][0] = accB[h][0] + wb[r][h][0] * me
                        accB[h][1] = accB[h][1] + wb[r][h][1] * mo
            for h in range(ng):
                sl = pl.ds(h * 32, 32)
                outb[i, sl] = plsc.pack(
                    accA[h][0], accA[h][1], format=_PK,
                    preferred_element_type=jnp.bfloat16)
                fe, fo = plsc.unpack(rows[rb + 48, sl], format=_PK,
                                     preferred_element_type=jnp.float32)
                outb[i, pl.ds(E + h * 32, 32)] = plsc.pack(
                    fe - accB[h][0], fo - accB[h][1], format=_PK,
                    preferred_element_type=jnp.bfloat16)
            return carry

        lax.fori_loop(0, ch, node, 0)

    ni = nch // 2

    def iteration(i, carry):
        c0 = 2 * i
        c1 = 2 * i + 1
        wait_g(rA, semA)

        @pl.when(i > 0)
        def _():
            wait_o(outA, semOA)

        compute(c0, rA, outA)
        pltpu.async_copy(outA, out_hbm.at[pl.ds(wbase + c0 * ch, ch)], semOA)

        @pl.when(i < ni - 1)
        def _():
            fire(c0 + 2, rA, semA)

        wait_g(rB, semB)

        @pl.when(i > 0)
        def _():
            wait_o(outB, semOB)

        compute(c1, rB, outB)
        pltpu.async_copy(outB, out_hbm.at[pl.ds(wbase + c1 * ch, ch)], semOB)

        @pl.when(i < ni - 1)
        def _():
            fire(c1 + 2, rB, semB)

        return carry

    lax.fori_loop(0, ni, iteration, 0)
    wait_o(outA, semOA)
    wait_o(outB, semOB)


def _agg_kernel(tbl, idxf, wv, *, E, total, per_w, ch, nch):
    tbl_rows = tbl.shape[0]
    body = functools.partial(_agg_body, E=E, per_w=per_w, ch=ch, nch=nch,
                             tbl_rows=tbl_rows)
    f = functools.partial(
        pl.kernel,
        out_type=jax.ShapeDtypeStruct((total, 2 * E), jnp.bfloat16),
        mesh=_mesh(),
        compiler_params=_SC_PARAMS,
        scratch_types=[
            pltpu.VMEM((per_w * R4,), jnp.int32),
            pltpu.VMEM_SHARED((tbl_rows, E), jnp.bfloat16),
            pltpu.VMEM((ch * R4, E), jnp.bfloat16),
            pltpu.VMEM((ch * R4, E), jnp.bfloat16),
            pltpu.VMEM((ch, 2 * E), jnp.bfloat16),
            pltpu.VMEM((ch, 2 * E), jnp.bfloat16),
            pltpu.VMEM((6 * E,), jnp.float32),
            pltpu.SemaphoreType.DMA,
            pltpu.SemaphoreType.DMA,
            pltpu.SemaphoreType.DMA,
            pltpu.SemaphoreType.DMA,
        ],
    )(body)
    return f(tbl, idxf, wv)


def _bg_body(nodes_hbm, aall_hbm, emb_hbm, int1_hbm,
             ab_hbm, e0b_hbm, i1b_hbm,
             nd_v, ab_v, e0b_v, i1b_v,
             s1, s2, s3):
    wid = _wid()
    base = wid * L2_PER_W
    pltpu.sync_copy(nodes_hbm.at[pl.ds(base, L2_PER_W)], nd_v)
    c1 = pltpu.async_copy(aall_hbm.at[nd_v], ab_v, s1)
    c2 = pltpu.async_copy(emb_hbm.at[nd_v], e0b_v, s2)
    c3 = pltpu.async_copy(int1_hbm.at[nd_v], i1b_v, s3)
    c1.wait(); c2.wait(); c3.wait()
    pltpu.sync_copy(ab_v, ab_hbm.at[pl.ds(base, L2_PER_W)])
    pltpu.sync_copy(e0b_v, e0b_hbm.at[pl.ds(base, L2_PER_W)])
    pltpu.sync_copy(i1b_v, i1b_hbm.at[pl.ds(base, L2_PER_W)])


def _bgather(nodes, aall64, emb0, inter1):
    f = functools.partial(
        pl.kernel,
        out_type=[jax.ShapeDtypeStruct((B, 64), jnp.int32),
                  jax.ShapeDtypeStruct((B, E1), jnp.bfloat16),
                  jax.ShapeDtypeStruct((B, E2), jnp.bfloat16)],
        mesh=_mesh(),
        compiler_params=_SC_PARAMS,
        scratch_types=[
            pltpu.VMEM((L2_PER_W,), jnp.int32),
            pltpu.VMEM((L2_PER_W, 64), jnp.int32),
            pltpu.VMEM((L2_PER_W, E1), jnp.bfloat16),
            pltpu.VMEM((L2_PER_W, E2), jnp.bfloat16),
            pltpu.SemaphoreType.DMA,
            pltpu.SemaphoreType.DMA,
            pltpu.SemaphoreType.DMA,
        ],
    )(_bg_body)
    return f(nodes, aall64, emb0, inter1)


def _head_body(e_ref, i1_ref, i2_ref, w2a, w2b, w2c, b2_ref, lp_ref, w3_ref,
               b3_ref, o_ref):
    x = (jnp.dot(e_ref[...].astype(jnp.float32), w2a[...],
                 preferred_element_type=jnp.float32)
         + jnp.dot(i1_ref[...].astype(jnp.float32), w2b[...],
                   preferred_element_type=jnp.float32)
         + jnp.dot(i2_ref[...].astype(jnp.float32), w2c[...],
                   preferred_element_type=jnp.float32)
         + b2_ref[...])
    x = jnp.where(x >= 0.0, x, 0.3 * x)
    x = x + lp_ref[...]
    y = jnp.dot(x, w3_ref[...], preferred_element_type=jnp.float32) + b3_ref[...]
    o_ref[...] = jax.nn.sigmoid(y)


def _head(e0b, i1b, inter2, W2, b2, prior, W3, b3):
    return pl.pallas_call(
        _head_body,
        out_shape=jax.ShapeDtypeStruct((B, 1), jnp.float32),
    )(e0b, i1b, inter2, W2[:E1], W2[E1:E1 + E2], W2[E1 + E2:],
      b2.reshape(1, 2), jnp.log(prior).reshape(1, 2), W3, b3.reshape(1, 1))


def kernel(nodes, feat_data, adj1, adj2, adj3, prior, W_mlp, b_mlp,
           alpha1, alpha2, W2, b2, W3, b3):
    emb0 = _emb(feat_data, W_mlp, b_mlp.reshape(1, E1))

    Wm1 = jax.nn.softmax(alpha1, axis=1)
    w1 = jnp.concatenate(
        [jnp.concatenate([_perm_w(Wm1[:E1, r]) for r in range(3)]),
         jnp.concatenate([_perm_w(Wm1[E1:, r]) for r in range(3)])])
    Wm2 = jax.nn.softmax(alpha2, axis=1)
    w2v = jnp.concatenate(
        [jnp.concatenate([_perm_w(Wm2[:E2, r]) for r in range(3)]),
         jnp.concatenate([_perm_w(Wm2[E2:, r]) for r in range(3)])])

    pad = ((0, NPAD - N), (0, 0))
    aall2d = jnp.concatenate(
        [jnp.pad(adj1, pad)[:, None, :], jnp.pad(adj2, pad)[:, None, :],
         jnp.pad(adj3, pad)[:, None, :]], axis=1).reshape(NPAD, 3 * DEG)
    iota = jnp.arange(NPAD, dtype=jnp.int32)[:, None]
    aall49 = jnp.concatenate([aall2d, iota], axis=1)
    aallf = aall49.reshape(-1)
    aall64 = jnp.concatenate(
        [aall49, jnp.broadcast_to(iota, (NPAD, 15))], axis=1)

    inter1 = _agg_kernel(emb0, aallf, w1, E=E1, total=NPAD,
                         per_w=L1_PER_W, ch=L1_CH, nch=L1_NCH)

    ab, e0b, i1b = _bgather(nodes, aall64, emb0, inter1)

    inter2 = _agg_kernel(inter1, ab[:, :R4].reshape(-1), w2v, E=E2, total=B,
                         per_w=L2_PER_W, ch=L2_CH, nch=L2_NCH)

    return _head(e0b, i1b, inter2, W2, b2, prior, W3, b3)
